# Initial kernel scaffold; baseline (speedup 1.0000x reference)
import jax
import jax.numpy as jnp
from jax.experimental import pallas as pl


def kernel(x, token_emb):
    raise NotImplementedError("write your pallas kernel here")


# speedup vs baseline: 1.5834x; 1.5834x over previous
import functools

import jax
import jax.numpy as jnp
from jax import lax
from jax.experimental import pallas as pl
from jax.experimental.pallas import tpu as pltpu
from jax.experimental.pallas import tpu_sc as plsc

B_TOTAL = 4 * 4096
DIM = 1024
NUM_WORKERS = 32
B_PER_W = B_TOTAL // NUM_WORKERS
CHUNK = 32
N_CHUNKS = B_PER_W // CHUNK

_mesh = plsc.VectorSubcoreMesh(core_axis_name="c", subcore_axis_name="s")


@functools.partial(
    pl.kernel,
    mesh=_mesh,
    out_type=jax.ShapeDtypeStruct((B_TOTAL, DIM), jnp.float32),
    scratch_types=[
        pltpu.VMEM((N_CHUNKS, CHUNK), jnp.int32),
        pltpu.VMEM((CHUNK, DIM), jnp.float32),
        pltpu.VMEM((CHUNK, DIM), jnp.float32),
        pltpu.SemaphoreType.DMA,
        pltpu.SemaphoreType.DMA,
        pltpu.SemaphoreType.DMA,
        pltpu.SemaphoreType.DMA,
    ],
)
def _emb_gather(idx_hbm, table_hbm, out_hbm, idx_v, buf0, buf1, gsem0, gsem1,
                osem0, osem1):
    wid = lax.axis_index("s") * 2 + lax.axis_index("c")
    base = wid * B_PER_W

    pltpu.sync_copy(idx_hbm.at[wid], idx_v)

    bufs = (buf0, buf1)
    gsems = (gsem0, gsem1)
    osems = (osem0, osem1)

    pltpu.async_copy(table_hbm.at[idx_v.at[0]], buf0, gsem0)

    def body(g, _):
        slot = lax.rem(g, 2)
        nslot = lax.rem(g + 1, 2)

        def run(sl, nsl):
            buf, gsem, osem = bufs[sl], gsems[sl], osems[sl]
            nbuf, ngsem, nosem = bufs[nsl], gsems[nsl], osems[nsl]
            pltpu.make_async_copy(table_hbm.at[idx_v.at[g]], buf, gsem).wait()
            @pl.when(g + 1 < N_CHUNKS)
            def _():
                @pl.when(g >= 1)
                def _():
                    pltpu.make_async_copy(
                        nbuf, out_hbm.at[pl.ds(base + (g - 1) * CHUNK, CHUNK)],
                        nosem).wait()
                pltpu.async_copy(table_hbm.at[idx_v.at[g + 1]], nbuf, ngsem)
            pltpu.async_copy(buf, out_hbm.at[pl.ds(base + g * CHUNK, CHUNK)],
                             osem)

        @pl.when(slot == 0)
        def _():
            run(0, 1)

        @pl.when(slot == 1)
        def _():
            run(1, 0)

        return 0

    lax.fori_loop(0, N_CHUNKS, body, 0)

    pltpu.make_async_copy(
        bufs[(N_CHUNKS - 2) % 2],
        out_hbm.at[pl.ds(base + (N_CHUNKS - 2) * CHUNK, CHUNK)],
        osems[(N_CHUNKS - 2) % 2]).wait()
    pltpu.make_async_copy(
        bufs[(N_CHUNKS - 1) % 2],
        out_hbm.at[pl.ds(base + (N_CHUNKS - 1) * CHUNK, CHUNK)],
        osems[(N_CHUNKS - 1) % 2]).wait()


def kernel(x, token_emb):
    idx = x.reshape(NUM_WORKERS, N_CHUNKS, CHUNK).astype(jnp.int32)
    out = _emb_gather(idx, token_emb)
    return out.reshape(x.shape + (DIM,))


# speedup vs baseline: 1.6424x; 1.0373x over previous
import functools

import jax
import jax.numpy as jnp
from jax import lax
from jax.experimental import pallas as pl
from jax.experimental.pallas import tpu as pltpu
from jax.experimental.pallas import tpu_sc as plsc

B_TOTAL = 4 * 4096
DIM = 1024
NUM_WORKERS = 32
B_PER_W = B_TOTAL // NUM_WORKERS
CHUNK = 32
N_CHUNKS = B_PER_W // CHUNK

NBUF = 3

_mesh = plsc.VectorSubcoreMesh(core_axis_name="c", subcore_axis_name="s")


@functools.partial(
    pl.kernel,
    mesh=_mesh,
    out_type=jax.ShapeDtypeStruct((B_TOTAL, DIM), jnp.float32),
    scratch_types=[
        pltpu.VMEM((N_CHUNKS, CHUNK), jnp.int32),
        *[pltpu.VMEM((CHUNK, DIM), jnp.float32) for _ in range(NBUF)],
        *[pltpu.SemaphoreType.DMA for _ in range(2 * NBUF)],
    ],
)
def _emb_gather(idx_hbm, table_hbm, out_hbm, idx_v, *scratch):
    bufs = scratch[:NBUF]
    gsems = scratch[NBUF:2 * NBUF]
    osems = scratch[2 * NBUF:]

    wid = lax.axis_index("s") * 2 + lax.axis_index("c")
    base = wid * B_PER_W

    pltpu.sync_copy(idx_hbm.at[wid], idx_v)

    def out_slice(g):
        return out_hbm.at[pl.ds(base + g * CHUNK, CHUNK)]

    for g in range(NBUF):
        pltpu.async_copy(table_hbm.at[idx_v.at[g]], bufs[g], gsems[g])

    for g in range(N_CHUNKS):
        b = g % NBUF
        pltpu.make_async_copy(table_hbm.at[idx_v.at[g]], bufs[b],
                              gsems[b]).wait()
        pltpu.async_copy(bufs[b], out_slice(g), osems[b])
        if g + NBUF < N_CHUNKS:
            pltpu.make_async_copy(bufs[b], out_slice(g), osems[b]).wait()
            pltpu.async_copy(table_hbm.at[idx_v.at[g + NBUF]], bufs[b],
                             gsems[b])

    for g in range(max(0, N_CHUNKS - NBUF), N_CHUNKS):
        pltpu.make_async_copy(bufs[g % NBUF], out_slice(g),
                              osems[g % NBUF]).wait()


def kernel(x, token_emb):
    idx = x.reshape(NUM_WORKERS, N_CHUNKS, CHUNK).astype(jnp.int32)
    out = _emb_gather(idx, token_emb)
    return out.reshape(x.shape + (DIM,))
